# baseline (speedup 1.0000x reference)
import functools

import jax
import jax.numpy as jnp
import numpy as np
from jax import lax
from jax.experimental import pallas as pl
from jax.experimental.pallas import tpu as pltpu
from jax.experimental.pallas import tpu_sc as plsc

_VOCAB = 100000
_EMB = 64
_BATCH = 1024
_SEQLEN = 200

_NC = 2
_NS = 16
_NW = _NC * _NS
_TOKENS = _BATCH * _SEQLEN
_PER_W = _TOKENS // _NW
_CH = 128
_NCH = _PER_W // _CH


def _pos_plus_seg(seg_table):
    pos = jnp.arange(_SEQLEN, dtype=jnp.float32)[:, None]
    d = jnp.arange(_EMB, dtype=jnp.float32)
    d = 2.0 * d / _EMB
    p = pos / jnp.power(10000.0, d)
    p = p.at[:, ::2].set(jnp.sin(p[:, ::2]))
    p = p.at[:, 1::2].set(jnp.cos(p[:, 1::2]))
    seg_sel = (jnp.arange(_SEQLEN) >= _SEQLEN // 2 + 1)[:, None]
    seg = jnp.where(seg_sel, seg_table[1][None, :], seg_table[0][None, :])
    return p + seg


_GATHER_DNUMS = lax.GatherDimensionNumbers(
    offset_dims=(), collapsed_slice_dims=(0,), start_index_map=(0,))
def _bfly_perms():
    lane = lax.iota(jnp.int32, 16)
    return [(lane ^ k).reshape(16, 1) for k in (1, 2, 4, 8)]


def _lane_sum(x, perms):
    for idx in perms:
        x = x + lax.gather(x, idx, dimension_numbers=_GATHER_DNUMS,
                           slice_sizes=(1,),
                           mode=lax.GatherScatterMode.PROMISE_IN_BOUNDS)
    return x


def _sc_body(idx_hbm, table_hbm, cmb_hbm, g_hbm, b_hbm, out_hbm,
             idx_v, rows_v, out_v, cmb_v, g_v, b_v, sem):
    wid = lax.axis_index("s") * _NC + lax.axis_index("c")
    pltpu.sync_copy(cmb_hbm, cmb_v)
    pltpu.sync_copy(g_hbm, g_v)
    pltpu.sync_copy(b_hbm, b_v)
    base = wid * _PER_W

    g_regs = [g_v[pl.ds(16 * j, 16)] for j in range(4)]
    b_regs = [b_v[pl.ds(16 * j, 16)] for j in range(4)]
    perms = _bfly_perms()
    pltpu.sync_copy(idx_hbm.at[pl.ds(wid * _NCH, _NCH)], idx_v)

    def chunk_body(c, carry):
        flat = base + c * _CH
        cbase = c * _CH
        with jax.named_scope("gth"):
            pltpu.async_copy(table_hbm.at[idx_v.at[c]], rows_v, sem).wait()

        def row_body(i, rcarry):
            for k in range(4):
                r = i * 4 + k
                p = lax.rem(cbase + r, _SEQLEN)
                v = [rows_v[r, pl.ds(16 * j, 16)] + cmb_v[p, pl.ds(16 * j, 16)]
                     for j in range(4)]
                s = (v[0] + v[1]) + (v[2] + v[3])
                q = ((v[0] * v[0] + v[1] * v[1]) + (v[2] * v[2] + v[3] * v[3]))
                mean = _lane_sum(s, perms) * (1.0 / 64.0)
                varv = _lane_sum(q, perms) * (1.0 / 64.0) - mean * mean + 1e-5
                iv = lax.bitcast_convert_type(varv, jnp.int32)
                y = lax.bitcast_convert_type(
                    jnp.int32(0x5F375A86) - lax.shift_right_logical(iv, 1),
                    jnp.float32)
                xh = varv * 0.5
                for _ in range(2):
                    y = y * (1.5 - xh * y * y)
                for j in range(4):
                    out_v[r, pl.ds(16 * j, 16)] = (
                        (v[j] - mean) * y * g_regs[j] + b_regs[j])
            return rcarry

        with jax.named_scope("cmp"):
            lax.fori_loop(0, _CH // 4, row_body, 0)
        with jax.named_scope("owr"):
            pltpu.sync_copy(out_v, out_hbm.at[pl.ds(flat, _CH)])
        return carry

    lax.fori_loop(0, _NCH, chunk_body, 0)


def kernel(seq, token_table, seg_table, gamma, beta):
    cmb = _pos_plus_seg(seg_table)
    idx = seq.reshape(_NW * _NCH, _CH)
    run = functools.partial(
        pl.kernel,
        out_type=jax.ShapeDtypeStruct((_TOKENS, _EMB), jnp.float32),
        mesh=plsc.VectorSubcoreMesh(core_axis_name="c", subcore_axis_name="s"),
        scratch_types=[
            pltpu.VMEM((_NCH, _CH), jnp.int32),
            pltpu.VMEM((_CH, _EMB), jnp.float32),
            pltpu.VMEM((_CH, _EMB), jnp.float32),
            pltpu.VMEM((_SEQLEN, _EMB), jnp.float32),
            pltpu.VMEM((_EMB,), jnp.float32),
            pltpu.VMEM((_EMB,), jnp.float32),
            pltpu.SemaphoreType.DMA,
        ],
        compiler_params=pltpu.CompilerParams(use_tc_tiling_on_sc=False),
    )(_sc_body)
    out = run(idx, token_table, cmb, gamma, beta)
    return out.reshape(_BATCH, _SEQLEN, _EMB)


# speedup vs baseline: 1.3810x; 1.3810x over previous
import functools

import jax
import jax.numpy as jnp
import numpy as np
from jax import lax
from jax.experimental import pallas as pl
from jax.experimental.pallas import tpu as pltpu
from jax.experimental.pallas import tpu_sc as plsc

_VOCAB = 100000
_EMB = 64
_BATCH = 1024
_SEQLEN = 200

_NC = 2
_NS = 16
_NW = _NC * _NS
_TOKENS = _BATCH * _SEQLEN
_PER_W = _TOKENS // _NW
_CH = 128
_NCH = _PER_W // _CH


def _pos_plus_seg(seg_table):
    pos = jnp.arange(_SEQLEN, dtype=jnp.float32)[:, None]
    d = jnp.arange(_EMB, dtype=jnp.float32)
    d = 2.0 * d / _EMB
    p = pos / jnp.power(10000.0, d)
    p = p.at[:, ::2].set(jnp.sin(p[:, ::2]))
    p = p.at[:, 1::2].set(jnp.cos(p[:, 1::2]))
    seg_sel = (jnp.arange(_SEQLEN) >= _SEQLEN // 2 + 1)[:, None]
    seg = jnp.where(seg_sel, seg_table[1][None, :], seg_table[0][None, :])
    return p + seg


_GATHER_DNUMS = lax.GatherDimensionNumbers(
    offset_dims=(), collapsed_slice_dims=(0,), start_index_map=(0,))
def _bfly_perms():
    lane = lax.iota(jnp.int32, 16)
    return [(lane ^ k).reshape(16, 1) for k in (1, 2, 4, 8)]


def _lane_sum(x, perms):
    for idx in perms:
        x = x + lax.gather(x, idx, dimension_numbers=_GATHER_DNUMS,
                           slice_sizes=(1,),
                           mode=lax.GatherScatterMode.PROMISE_IN_BOUNDS)
    return x


def _sc_body(idx_hbm, table_hbm, cmb_hbm, g_hbm, b_hbm, out_hbm,
             idx_v, rows_v, out_v, cmb_v, g_v, b_v, sem):
    wid = lax.axis_index("s") * _NC + lax.axis_index("c")
    pltpu.sync_copy(cmb_hbm, cmb_v)
    pltpu.sync_copy(g_hbm, g_v)
    pltpu.sync_copy(b_hbm, b_v)
    base = wid * _PER_W

    g_regs = [g_v[pl.ds(16 * j, 16)] for j in range(4)]
    b_regs = [b_v[pl.ds(16 * j, 16)] for j in range(4)]
    perms = _bfly_perms()
    pltpu.sync_copy(idx_hbm.at[pl.ds(wid * _NCH, _NCH)], idx_v)

    def chunk_body(c, carry):
        flat = base + c * _CH
        cbase = c * _CH
        with jax.named_scope("gth"):
            pltpu.async_copy(table_hbm.at[idx_v.at[c]], rows_v, sem).wait()

        def row_body(i, rcarry):
            vs = []
            for k in range(4):
                r = i * 4 + k
                p = lax.rem(cbase + r, _SEQLEN)
                vs.append([rows_v[r, pl.ds(16 * j, 16)]
                           + cmb_v[p, pl.ds(16 * j, 16)] for j in range(4)])
            ys, means = [], []
            for k in range(4):
                v = vs[k]
                s = (v[0] + v[1]) + (v[2] + v[3])
                q = ((v[0] * v[0] + v[1] * v[1])
                     + (v[2] * v[2] + v[3] * v[3]))
                mean = _lane_sum(s, perms) * (1.0 / 64.0)
                varv = _lane_sum(q, perms) * (1.0 / 64.0) - mean * mean + 1e-5
                iv = lax.bitcast_convert_type(varv, jnp.int32)
                y = lax.bitcast_convert_type(
                    jnp.int32(0x5F375A86) - lax.shift_right_logical(iv, 1),
                    jnp.float32)
                xh = varv * 0.5
                for _ in range(2):
                    y = y * (1.5 - xh * y * y)
                ys.append(y)
                means.append(mean)
            for k in range(4):
                r = i * 4 + k
                for j in range(4):
                    out_v[r, pl.ds(16 * j, 16)] = (
                        (vs[k][j] - means[k]) * ys[k] * g_regs[j] + b_regs[j])
            return rcarry

        with jax.named_scope("cmp"):
            lax.fori_loop(0, _CH // 4, row_body, 0)
        with jax.named_scope("owr"):
            pltpu.sync_copy(out_v, out_hbm.at[pl.ds(flat, _CH)])
        return carry

    lax.fori_loop(0, _NCH, chunk_body, 0)


def kernel(seq, token_table, seg_table, gamma, beta):
    cmb = _pos_plus_seg(seg_table)
    idx = seq.reshape(_NW * _NCH, _CH)
    run = functools.partial(
        pl.kernel,
        out_type=jax.ShapeDtypeStruct((_TOKENS, _EMB), jnp.float32),
        mesh=plsc.VectorSubcoreMesh(core_axis_name="c", subcore_axis_name="s"),
        scratch_types=[
            pltpu.VMEM((_NCH, _CH), jnp.int32),
            pltpu.VMEM((_CH, _EMB), jnp.float32),
            pltpu.VMEM((_CH, _EMB), jnp.float32),
            pltpu.VMEM((_SEQLEN, _EMB), jnp.float32),
            pltpu.VMEM((_EMB,), jnp.float32),
            pltpu.VMEM((_EMB,), jnp.float32),
            pltpu.SemaphoreType.DMA,
        ],
        compiler_params=pltpu.CompilerParams(use_tc_tiling_on_sc=False),
    )(_sc_body)
    out = run(idx, token_table, cmb, gamma, beta)
    return out.reshape(_BATCH, _SEQLEN, _EMB)


# speedup vs baseline: 1.3983x; 1.0125x over previous
import functools

import jax
import jax.numpy as jnp
import numpy as np
from jax import lax
from jax.experimental import pallas as pl
from jax.experimental.pallas import tpu as pltpu
from jax.experimental.pallas import tpu_sc as plsc

_VOCAB = 100000
_EMB = 64
_BATCH = 1024
_SEQLEN = 200

_NC = 2
_NS = 16
_NW = _NC * _NS
_TOKENS = _BATCH * _SEQLEN
_PER_W = _TOKENS // _NW
_CH = 128
_NCH = _PER_W // _CH


def _pos_plus_seg(seg_table):
    pos = jnp.arange(_SEQLEN, dtype=jnp.float32)[:, None]
    d = jnp.arange(_EMB, dtype=jnp.float32)
    d = 2.0 * d / _EMB
    p = pos / jnp.power(10000.0, d)
    p = p.at[:, ::2].set(jnp.sin(p[:, ::2]))
    p = p.at[:, 1::2].set(jnp.cos(p[:, 1::2]))
    seg_sel = (jnp.arange(_SEQLEN) >= _SEQLEN // 2 + 1)[:, None]
    seg = jnp.where(seg_sel, seg_table[1][None, :], seg_table[0][None, :])
    return p + seg


_GATHER_DNUMS = lax.GatherDimensionNumbers(
    offset_dims=(), collapsed_slice_dims=(0,), start_index_map=(0,))
def _bfly_perms():
    lane = lax.iota(jnp.int32, 16)
    return [(lane ^ k).reshape(16, 1) for k in (1, 2, 4, 8)]


def _lane_sum(x, perms):
    for idx in perms:
        x = x + lax.gather(x, idx, dimension_numbers=_GATHER_DNUMS,
                           slice_sizes=(1,),
                           mode=lax.GatherScatterMode.PROMISE_IN_BOUNDS)
    return x


def _sc_body(idx_hbm, table_hbm, cmb_hbm, g_hbm, b_hbm, out_hbm,
             idx_v, rows_v, out_v, cmb_v, g_v, b_v, sem):
    wid = lax.axis_index("s") * _NC + lax.axis_index("c")
    pltpu.sync_copy(cmb_hbm, cmb_v)
    pltpu.sync_copy(g_hbm, g_v)
    pltpu.sync_copy(b_hbm, b_v)
    base = wid * _PER_W

    g_regs = [g_v[pl.ds(16 * j, 16)] for j in range(4)]
    b_regs = [b_v[pl.ds(16 * j, 16)] for j in range(4)]
    perms = _bfly_perms()
    pltpu.sync_copy(idx_hbm.at[pl.ds(wid * _NCH, _NCH)], idx_v)

    def chunk_body(c, carry):
        flat = base + c * _CH
        cbase = c * _CH
        with jax.named_scope("gth"):
            pltpu.async_copy(table_hbm.at[idx_v.at[c]], rows_v, sem).wait()

        def row_body(i, rcarry):
            vs = []
            for k in range(8):
                r = i * 8 + k
                p = lax.rem(cbase + r, _SEQLEN)
                vs.append([rows_v[r, pl.ds(16 * j, 16)]
                           + cmb_v[p, pl.ds(16 * j, 16)] for j in range(4)])
            ys, means = [], []
            for k in range(8):
                v = vs[k]
                s = (v[0] + v[1]) + (v[2] + v[3])
                q = ((v[0] * v[0] + v[1] * v[1])
                     + (v[2] * v[2] + v[3] * v[3]))
                mean = _lane_sum(s, perms) * (1.0 / 64.0)
                varv = _lane_sum(q, perms) * (1.0 / 64.0) - mean * mean + 1e-5
                iv = lax.bitcast_convert_type(varv, jnp.int32)
                y = lax.bitcast_convert_type(
                    jnp.int32(0x5F375A86) - lax.shift_right_logical(iv, 1),
                    jnp.float32)
                xh = varv * 0.5
                for _ in range(2):
                    y = y * (1.5 - xh * y * y)
                ys.append(y)
                means.append(mean)
            for k in range(8):
                r = i * 8 + k
                for j in range(4):
                    out_v[r, pl.ds(16 * j, 16)] = (
                        (vs[k][j] - means[k]) * ys[k] * g_regs[j] + b_regs[j])
            return rcarry

        with jax.named_scope("cmp"):
            lax.fori_loop(0, _CH // 8, row_body, 0)
        with jax.named_scope("owr"):
            pltpu.sync_copy(out_v, out_hbm.at[pl.ds(flat, _CH)])
        return carry

    lax.fori_loop(0, _NCH, chunk_body, 0)


def kernel(seq, token_table, seg_table, gamma, beta):
    cmb = _pos_plus_seg(seg_table)
    idx = seq.reshape(_NW * _NCH, _CH)
    run = functools.partial(
        pl.kernel,
        out_type=jax.ShapeDtypeStruct((_TOKENS, _EMB), jnp.float32),
        mesh=plsc.VectorSubcoreMesh(core_axis_name="c", subcore_axis_name="s"),
        scratch_types=[
            pltpu.VMEM((_NCH, _CH), jnp.int32),
            pltpu.VMEM((_CH, _EMB), jnp.float32),
            pltpu.VMEM((_CH, _EMB), jnp.float32),
            pltpu.VMEM((_SEQLEN, _EMB), jnp.float32),
            pltpu.VMEM((_EMB,), jnp.float32),
            pltpu.VMEM((_EMB,), jnp.float32),
            pltpu.SemaphoreType.DMA,
        ],
        compiler_params=pltpu.CompilerParams(use_tc_tiling_on_sc=False),
    )(_sc_body)
    out = run(idx, token_table, cmb, gamma, beta)
    return out.reshape(_BATCH, _SEQLEN, _EMB)


# speedup vs baseline: 1.6918x; 1.2099x over previous
import functools

import jax
import jax.numpy as jnp
import numpy as np
from jax import lax
from jax.experimental import pallas as pl
from jax.experimental.pallas import tpu as pltpu
from jax.experimental.pallas import tpu_sc as plsc

_VOCAB = 100000
_EMB = 64
_BATCH = 1024
_SEQLEN = 200

_NC = 2
_NS = 16
_NW = _NC * _NS
_TOKENS = _BATCH * _SEQLEN
_PER_W = _TOKENS // _NW
_CH = 128
_NCH = _PER_W // _CH


def _pos_plus_seg(seg_table):
    pos = jnp.arange(_SEQLEN, dtype=jnp.float32)[:, None]
    d = jnp.arange(_EMB, dtype=jnp.float32)
    d = 2.0 * d / _EMB
    p = pos / jnp.power(10000.0, d)
    p = p.at[:, ::2].set(jnp.sin(p[:, ::2]))
    p = p.at[:, 1::2].set(jnp.cos(p[:, 1::2]))
    seg_sel = (jnp.arange(_SEQLEN) >= _SEQLEN // 2 + 1)[:, None]
    seg = jnp.where(seg_sel, seg_table[1][None, :], seg_table[0][None, :])
    return p + seg


_GATHER_DNUMS = lax.GatherDimensionNumbers(
    offset_dims=(), collapsed_slice_dims=(0,), start_index_map=(0,))
def _bfly_perms():
    lane = lax.iota(jnp.int32, 16)
    return [(lane ^ k).reshape(16, 1) for k in (1, 2, 4, 8)]


def _lane_sum(x, perms):
    for idx in perms:
        x = x + lax.gather(x, idx, dimension_numbers=_GATHER_DNUMS,
                           slice_sizes=(1,),
                           mode=lax.GatherScatterMode.PROMISE_IN_BOUNDS)
    return x


def _sc_body(idx_hbm, table_hbm, cmb_hbm, g_hbm, b_hbm, out_hbm,
             idx_v, rows0, rows1, outv0, outv1, cmb_v, g_v, b_v,
             sg0, sg1, so0, so1):
    wid = lax.axis_index("s") * _NC + lax.axis_index("c")
    pltpu.sync_copy(cmb_hbm, cmb_v)
    pltpu.sync_copy(g_hbm, g_v)
    pltpu.sync_copy(b_hbm, b_v)
    base = wid * _PER_W
    rows = [rows0, rows1]
    outs = [outv0, outv1]
    sg = [sg0, sg1]
    so = [so0, so1]

    g_regs = [g_v[pl.ds(16 * j, 16)] for j in range(4)]
    b_regs = [b_v[pl.ds(16 * j, 16)] for j in range(4)]
    perms = _bfly_perms()
    pltpu.sync_copy(idx_hbm.at[pl.ds(wid * _NCH, _NCH)], idx_v)

    def compute(cbase, rows_v, out_v):
        def row_body(i, rcarry):
            vs = []
            for k in range(8):
                r = i * 8 + k
                p = lax.rem(cbase + r, _SEQLEN)
                vs.append([rows_v[r, pl.ds(16 * j, 16)]
                           + cmb_v[p, pl.ds(16 * j, 16)] for j in range(4)])
            ys, means = [], []
            for k in range(8):
                v = vs[k]
                s = (v[0] + v[1]) + (v[2] + v[3])
                q = ((v[0] * v[0] + v[1] * v[1])
                     + (v[2] * v[2] + v[3] * v[3]))
                mean = _lane_sum(s, perms) * (1.0 / 64.0)
                varv = _lane_sum(q, perms) * (1.0 / 64.0) - mean * mean + 1e-5
                iv = lax.bitcast_convert_type(varv, jnp.int32)
                y = lax.bitcast_convert_type(
                    jnp.int32(0x5F375A86) - lax.shift_right_logical(iv, 1),
                    jnp.float32)
                xh = varv * 0.5
                for _ in range(2):
                    y = y * (1.5 - xh * y * y)
                ys.append(y)
                means.append(mean)
            for k in range(8):
                r = i * 8 + k
                for j in range(4):
                    out_v[r, pl.ds(16 * j, 16)] = (
                        (vs[k][j] - means[k]) * ys[k] * g_regs[j] + b_regs[j])
            return rcarry

        lax.fori_loop(0, _CH // 8, row_body, 0)

    def do_chunk(c, b, prefetch, outwait):
        flat = base + c * _CH
        pltpu.make_async_copy(table_hbm.at[idx_v.at[c]], rows[b], sg[b]).wait()
        if prefetch is not None:
            pltpu.async_copy(
                table_hbm.at[idx_v.at[prefetch]], rows[1 - b], sg[1 - b])
        if outwait:
            pltpu.make_async_copy(
                outs[b], out_hbm.at[pl.ds(flat, _CH)], so[b]).wait()
        compute(c * _CH, rows[b], outs[b])
        pltpu.async_copy(outs[b], out_hbm.at[pl.ds(flat, _CH)], so[b])

    pltpu.async_copy(table_hbm.at[idx_v.at[0]], rows[0], sg[0])
    do_chunk(0, 0, 1, False)
    do_chunk(1, 1, 2, False)

    def grp(g, carry):
        c0 = 2 * g + 2
        do_chunk(c0, 0, c0 + 1, True)
        do_chunk(c0 + 1, 1, c0 + 2, True)
        return carry

    lax.fori_loop(0, (_NCH - 4) // 2, grp, 0)
    do_chunk(_NCH - 2, 0, _NCH - 1, True)
    do_chunk(_NCH - 1, 1, None, True)
    pltpu.make_async_copy(
        outs[0], out_hbm.at[pl.ds(base + (_NCH - 2) * _CH, _CH)], so[0]).wait()
    pltpu.make_async_copy(
        outs[1], out_hbm.at[pl.ds(base + (_NCH - 1) * _CH, _CH)], so[1]).wait()


def kernel(seq, token_table, seg_table, gamma, beta):
    cmb = _pos_plus_seg(seg_table)
    idx = seq.reshape(_NW * _NCH, _CH)
    run = functools.partial(
        pl.kernel,
        out_type=jax.ShapeDtypeStruct((_TOKENS, _EMB), jnp.float32),
        mesh=plsc.VectorSubcoreMesh(core_axis_name="c", subcore_axis_name="s"),
        scratch_types=[
            pltpu.VMEM((_NCH, _CH), jnp.int32),
            pltpu.VMEM((_CH, _EMB), jnp.float32),
            pltpu.VMEM((_CH, _EMB), jnp.float32),
            pltpu.VMEM((_CH, _EMB), jnp.float32),
            pltpu.VMEM((_CH, _EMB), jnp.float32),
            pltpu.VMEM((_SEQLEN, _EMB), jnp.float32),
            pltpu.VMEM((_EMB,), jnp.float32),
            pltpu.VMEM((_EMB,), jnp.float32),
            pltpu.SemaphoreType.DMA,
            pltpu.SemaphoreType.DMA,
            pltpu.SemaphoreType.DMA,
            pltpu.SemaphoreType.DMA,
        ],
        compiler_params=pltpu.CompilerParams(use_tc_tiling_on_sc=False),
    )(_sc_body)
    out = run(idx, token_table, cmb, gamma, beta)
    return out.reshape(_BATCH, _SEQLEN, _EMB)


# speedup vs baseline: 2.1122x; 1.2485x over previous
import functools

import jax
import jax.numpy as jnp
from jax import lax
from jax.experimental import pallas as pl
from jax.experimental.pallas import tpu as pltpu
from jax.experimental.pallas import tpu_sc as plsc

_VOCAB = 100000
_EMB = 64
_BATCH = 1024
_SEQLEN = 200

_NC = 2
_NS = 16
_NW = _NC * _NS
_NBLK = _BATCH // 128
_SPW = _SEQLEN * _NBLK // _NW
_CH = 128


def _pos_plus_seg(seg_table):
    pos = jnp.arange(_SEQLEN, dtype=jnp.float32)[:, None]
    d = jnp.arange(_EMB, dtype=jnp.float32)
    d = 2.0 * d / _EMB
    p = pos / jnp.power(10000.0, d)
    p = p.at[:, ::2].set(jnp.sin(p[:, ::2]))
    p = p.at[:, 1::2].set(jnp.cos(p[:, 1::2]))
    seg_sel = (jnp.arange(_SEQLEN) >= _SEQLEN // 2 + 1)[:, None]
    seg = jnp.where(seg_sel, seg_table[1][None, :], seg_table[0][None, :])
    return p + seg


_GATHER_DNUMS = lax.GatherDimensionNumbers(
    offset_dims=(), collapsed_slice_dims=(0,), start_index_map=(0,))


def _bfly_perms():
    lane = lax.iota(jnp.int32, 16)
    return [(lane ^ k).reshape(16, 1) for k in (1, 2, 4, 8)]


def _lane_sum(x, perms):
    for idx in perms:
        x = x + lax.gather(x, idx, dimension_numbers=_GATHER_DNUMS,
                           slice_sizes=(1,),
                           mode=lax.GatherScatterMode.PROMISE_IN_BOUNDS)
    return x


def _sc_body(idx_hbm, table_hbm, cmb_hbm, g_hbm, b_hbm, out_hbm,
             idx_v, rows0, rows1, ob0, ob1, cmb_v, g_v, b_v,
             sg0, sg1, so0, so1):
    wid = lax.axis_index("s") * _NC + lax.axis_index("c")
    blk = lax.rem(wid, _NBLK)
    s0 = lax.div(wid, _NBLK) * _SPW
    pltpu.sync_copy(cmb_hbm.at[pl.ds(s0, _SPW)], cmb_v)
    pltpu.sync_copy(g_hbm, g_v)
    pltpu.sync_copy(b_hbm, b_v)
    pltpu.sync_copy(idx_hbm.at[pl.ds(s0, _SPW)], idx_v)
    rows = [rows0, rows1]
    outs = [ob0, ob1]
    sg = [sg0, sg1]
    so = [so0, so1]

    g_regs = [g_v[pl.ds(16 * j, 16)] for j in range(4)]
    b_regs = [b_v[pl.ds(16 * j, 16)] for j in range(4)]
    perms = _bfly_perms()
    lane = lax.iota(jnp.int32, 16)
    tmasks = [lax.bitwise_and(lane, t) == 0 for t in (1, 2, 4, 8)]

    def transpose16(regs):
        for ti, t in enumerate((1, 2, 4, 8)):
            m = tmasks[ti]
            new = []
            for k in range(16):
                other = lax.gather(
                    regs[k ^ t], perms[ti], dimension_numbers=_GATHER_DNUMS,
                    slice_sizes=(1,),
                    mode=lax.GatherScatterMode.PROMISE_IN_BOUNDS)
                if k & t == 0:
                    new.append(jnp.where(m, regs[k], other))
                else:
                    new.append(jnp.where(m, other, regs[k]))
            regs = new
        return regs

    def compute(c, rows_v, out_b):
        cmb_regs = [cmb_v[c, pl.ds(16 * j, 16)] for j in range(4)]

        def row_body(i, rcarry):
            ys, means = [], []
            for half in range(2):
                vs = []
                for k in range(8):
                    r = i * 16 + half * 8 + k
                    vs.append([rows_v[r, pl.ds(16 * j, 16)] + cmb_regs[j]
                               for j in range(4)])
                for k in range(8):
                    v = vs[k]
                    s = (v[0] + v[1]) + (v[2] + v[3])
                    q = ((v[0] * v[0] + v[1] * v[1])
                         + (v[2] * v[2] + v[3] * v[3]))
                    mean = _lane_sum(s, perms) * (1.0 / 64.0)
                    varv = (_lane_sum(q, perms) * (1.0 / 64.0)
                            - mean * mean + 1e-5)
                    iv = lax.bitcast_convert_type(varv, jnp.int32)
                    y = lax.bitcast_convert_type(
                        jnp.int32(0x5F375A86) - lax.shift_right_logical(iv, 1),
                        jnp.float32)
                    xh = varv * 0.5
                    for _ in range(2):
                        y = y * (1.5 - xh * y * y)
                    ys.append(y)
                    means.append(mean)
            for jj in range(4):
                regs = []
                for k in range(16):
                    r = i * 16 + k
                    t = rows_v[r, pl.ds(16 * jj, 16)] + cmb_regs[jj]
                    regs.append((t - means[k]) * ys[k])
                regs = transpose16(regs)
                for l in range(16):
                    j = 16 * jj + l
                    out_b[pl.ds(j * 128 + i * 16, 16)] = (
                        regs[l] * g_regs[jj][l] + b_regs[jj][l])
            return rcarry

        lax.fori_loop(0, _CH // 16, row_body, 0)

    def do_chunk(c, b, prefetch, outwait):
        pltpu.make_async_copy(
            table_hbm.at[idx_v.at[c, blk]], rows[b], sg[b]).wait()
        if prefetch is not None:
            pltpu.async_copy(
                table_hbm.at[idx_v.at[prefetch, blk]], rows[1 - b],
                sg[1 - b])
        off = ((s0 + c) * 8 * _NBLK + blk) * 1024
        if outwait:
            for tr in range(8):
                pltpu.make_async_copy(
                    outs[b].at[pl.ds(tr * 1024, 1024)],
                    out_hbm.at[pl.ds(off + tr * _NBLK * 1024, 1024)],
                    so[b]).wait()
        compute(c, rows[b], outs[b])
        for tr in range(8):
            pltpu.async_copy(
                outs[b].at[pl.ds(tr * 1024, 1024)],
                out_hbm.at[pl.ds(off + tr * _NBLK * 1024, 1024)], so[b])

    pltpu.async_copy(table_hbm.at[idx_v.at[0, blk]], rows[0], sg[0])
    do_chunk(0, 0, 1, False)
    do_chunk(1, 1, 2, False)

    def grp(g, carry):
        c0 = 2 * g + 2
        do_chunk(c0, 0, c0 + 1, True)
        do_chunk(c0 + 1, 1, c0 + 2, True)
        return carry

    lax.fori_loop(0, (_SPW - 4) // 2, grp, 0)
    do_chunk(_SPW - 2, 0, _SPW - 1, True)
    do_chunk(_SPW - 1, 1, None, True)
    for b, c in ((0, _SPW - 2), (1, _SPW - 1)):
        off = ((s0 + c) * 8 * _NBLK + blk) * 1024
        for tr in range(8):
            pltpu.make_async_copy(
                outs[b].at[pl.ds(tr * 1024, 1024)],
                out_hbm.at[pl.ds(off + tr * _NBLK * 1024, 1024)],
                so[b]).wait()


def kernel(seq, token_table, seg_table, gamma, beta):
    cmb = _pos_plus_seg(seg_table)
    idx3 = jnp.transpose(seq, (1, 0)).reshape(_SEQLEN, _NBLK, 128)
    run = functools.partial(
        pl.kernel,
        out_type=jax.ShapeDtypeStruct((_SEQLEN * 8 * _NBLK * 8 * 128,),
                                      jnp.float32),
        mesh=plsc.VectorSubcoreMesh(core_axis_name="c", subcore_axis_name="s"),
        scratch_types=[
            pltpu.VMEM((_SPW, _NBLK, 128), jnp.int32),
            pltpu.VMEM((_CH, _EMB), jnp.float32),
            pltpu.VMEM((_CH, _EMB), jnp.float32),
            pltpu.VMEM((8192,), jnp.float32),
            pltpu.VMEM((8192,), jnp.float32),
            pltpu.VMEM((_SPW, _EMB), jnp.float32),
            pltpu.VMEM((_EMB,), jnp.float32),
            pltpu.VMEM((_EMB,), jnp.float32),
            pltpu.SemaphoreType.DMA,
            pltpu.SemaphoreType.DMA,
            pltpu.SemaphoreType.DMA,
            pltpu.SemaphoreType.DMA,
        ],
        compiler_params=pltpu.CompilerParams(use_tc_tiling_on_sc=False),
    )(_sc_body)
    outf = run(idx3, token_table, cmb, gamma, beta)
    out5 = outf.reshape(_SEQLEN, 8, _NBLK, 8, 128)
    return out5.transpose((2, 4, 0, 1, 3)).reshape(_BATCH, _SEQLEN, _EMB)


# speedup vs baseline: 2.3001x; 1.0890x over previous
import functools

import jax
import jax.numpy as jnp
from jax import lax
from jax.experimental import pallas as pl
from jax.experimental.pallas import tpu as pltpu
from jax.experimental.pallas import tpu_sc as plsc

_VOCAB = 100000
_EMB = 64
_BATCH = 1024
_SEQLEN = 200

_NC = 2
_NS = 16
_NW = _NC * _NS
_NBLK = _BATCH // 128
_SPW = _SEQLEN * _NBLK // _NW
_CH = 128


def _pos_plus_seg(seg_table):
    pos = jnp.arange(_SEQLEN, dtype=jnp.float32)[:, None]
    d = jnp.arange(_EMB, dtype=jnp.float32)
    d = 2.0 * d / _EMB
    p = pos / jnp.power(10000.0, d)
    p = p.at[:, ::2].set(jnp.sin(p[:, ::2]))
    p = p.at[:, 1::2].set(jnp.cos(p[:, 1::2]))
    seg_sel = (jnp.arange(_SEQLEN) >= _SEQLEN // 2 + 1)[:, None]
    seg = jnp.where(seg_sel, seg_table[1][None, :], seg_table[0][None, :])
    return p + seg


_GATHER_DNUMS = lax.GatherDimensionNumbers(
    offset_dims=(), collapsed_slice_dims=(0,), start_index_map=(0,))


def _bfly_perms():
    lane = lax.iota(jnp.int32, 16)
    return [(lane ^ k).reshape(16, 1) for k in (1, 2, 4, 8)]


def _lane_sum(x, perms):
    for idx in perms:
        x = x + lax.gather(x, idx, dimension_numbers=_GATHER_DNUMS,
                           slice_sizes=(1,),
                           mode=lax.GatherScatterMode.PROMISE_IN_BOUNDS)
    return x


def _sc_body(idx_hbm, table_hbm, cmb_hbm, g_hbm, b_hbm, out_hbm,
             idx_v, rows0, rows1, ob0, ob1, cmb_v, g_v, b_v,
             sg0, sg1, so0, so1):
    wid = lax.axis_index("s") * _NC + lax.axis_index("c")
    blk = lax.rem(wid, _NBLK)
    s0 = lax.div(wid, _NBLK) * _SPW
    pltpu.sync_copy(cmb_hbm.at[pl.ds(s0, _SPW)], cmb_v)
    pltpu.sync_copy(g_hbm, g_v)
    pltpu.sync_copy(b_hbm, b_v)
    pltpu.sync_copy(idx_hbm.at[pl.ds(s0, _SPW)], idx_v)
    rows = [rows0, rows1]
    outs = [ob0, ob1]
    sg = [sg0, sg1]
    so = [so0, so1]

    g_regs = [g_v[pl.ds(16 * j, 16)] for j in range(4)]
    b_regs = [b_v[pl.ds(16 * j, 16)] for j in range(4)]
    perms = _bfly_perms()
    lane = lax.iota(jnp.int32, 16)
    tmasks = [lax.bitwise_and(lane, t) == 0 for t in (1, 2, 4, 8)]

    def transpose16(regs):
        for ti, t in enumerate((1, 2, 4, 8)):
            m = tmasks[ti]
            new = []
            for k in range(16):
                other = lax.gather(
                    regs[k ^ t], perms[ti], dimension_numbers=_GATHER_DNUMS,
                    slice_sizes=(1,),
                    mode=lax.GatherScatterMode.PROMISE_IN_BOUNDS)
                if k & t == 0:
                    new.append(jnp.where(m, regs[k], other))
                else:
                    new.append(jnp.where(m, other, regs[k]))
            regs = new
        return regs

    def compute(c, rows_v, out_b):
        cmb_regs = [cmb_v[c, pl.ds(16 * j, 16)] for j in range(4)]

        def row_body(i, rcarry):
            acc_s = None
            acc_q = None
            for jj in range(4):
                regs = []
                for k in range(16):
                    r = i * 16 + k
                    regs.append(rows_v[r, pl.ds(16 * jj, 16)] + cmb_regs[jj])
                regs = transpose16(regs)
                for l in range(16):
                    t = regs[l]
                    if acc_s is None:
                        acc_s, acc_q = t, t * t
                    else:
                        acc_s = acc_s + t
                        acc_q = acc_q + t * t
                    out_b[pl.ds((16 * jj + l) * 128 + i * 16, 16)] = t
            mean = acc_s * (1.0 / 64.0)
            varv = acc_q * (1.0 / 64.0) - mean * mean + 1e-5
            iv = lax.bitcast_convert_type(varv, jnp.int32)
            y = lax.bitcast_convert_type(
                jnp.int32(0x5F375A86) - lax.shift_right_logical(iv, 1),
                jnp.float32)
            xh = varv * 0.5
            for _ in range(2):
                y = y * (1.5 - xh * y * y)
            off = mean * y
            for jj in range(4):
                for l in range(16):
                    j = 16 * jj + l
                    t = out_b[pl.ds(j * 128 + i * 16, 16)]
                    out_b[pl.ds(j * 128 + i * 16, 16)] = (
                        (t * y - off) * g_regs[jj][l] + b_regs[jj][l])
            return rcarry

        lax.fori_loop(0, _CH // 16, row_body, 0)

    def do_chunk(c, b, prefetch, outwait):
        pltpu.make_async_copy(
            table_hbm.at[idx_v.at[c, blk]], rows[b], sg[b]).wait()
        if prefetch is not None:
            pltpu.async_copy(
                table_hbm.at[idx_v.at[prefetch, blk]], rows[1 - b],
                sg[1 - b])
        off = ((s0 + c) * 8 * _NBLK + blk) * 1024
        if outwait:
            for tr in range(8):
                pltpu.make_async_copy(
                    outs[b].at[pl.ds(tr * 1024, 1024)],
                    out_hbm.at[pl.ds(off + tr * _NBLK * 1024, 1024)],
                    so[b]).wait()
        compute(c, rows[b], outs[b])
        for tr in range(8):
            pltpu.async_copy(
                outs[b].at[pl.ds(tr * 1024, 1024)],
                out_hbm.at[pl.ds(off + tr * _NBLK * 1024, 1024)], so[b])

    pltpu.async_copy(table_hbm.at[idx_v.at[0, blk]], rows[0], sg[0])
    do_chunk(0, 0, 1, False)
    do_chunk(1, 1, 2, False)

    def grp(g, carry):
        c0 = 2 * g + 2
        do_chunk(c0, 0, c0 + 1, True)
        do_chunk(c0 + 1, 1, c0 + 2, True)
        return carry

    lax.fori_loop(0, (_SPW - 4) // 2, grp, 0)
    do_chunk(_SPW - 2, 0, _SPW - 1, True)
    do_chunk(_SPW - 1, 1, None, True)
    for b, c in ((0, _SPW - 2), (1, _SPW - 1)):
        off = ((s0 + c) * 8 * _NBLK + blk) * 1024
        for tr in range(8):
            pltpu.make_async_copy(
                outs[b].at[pl.ds(tr * 1024, 1024)],
                out_hbm.at[pl.ds(off + tr * _NBLK * 1024, 1024)],
                so[b]).wait()


def kernel(seq, token_table, seg_table, gamma, beta):
    cmb = _pos_plus_seg(seg_table)
    idx3 = jnp.transpose(seq, (1, 0)).reshape(_SEQLEN, _NBLK, 128)
    run = functools.partial(
        pl.kernel,
        out_type=jax.ShapeDtypeStruct((_SEQLEN * 8 * _NBLK * 8 * 128,),
                                      jnp.float32),
        mesh=plsc.VectorSubcoreMesh(core_axis_name="c", subcore_axis_name="s"),
        scratch_types=[
            pltpu.VMEM((_SPW, _NBLK, 128), jnp.int32),
            pltpu.VMEM((_CH, _EMB), jnp.float32),
            pltpu.VMEM((_CH, _EMB), jnp.float32),
            pltpu.VMEM((8192,), jnp.float32),
            pltpu.VMEM((8192,), jnp.float32),
            pltpu.VMEM((_SPW, _EMB), jnp.float32),
            pltpu.VMEM((_EMB,), jnp.float32),
            pltpu.VMEM((_EMB,), jnp.float32),
            pltpu.SemaphoreType.DMA,
            pltpu.SemaphoreType.DMA,
            pltpu.SemaphoreType.DMA,
            pltpu.SemaphoreType.DMA,
        ],
        compiler_params=pltpu.CompilerParams(use_tc_tiling_on_sc=False),
    )(_sc_body)
    outf = run(idx3, token_table, cmb, gamma, beta)
    out5 = outf.reshape(_SEQLEN, 8, _NBLK, 8, 128)
    return out5.transpose((2, 4, 0, 1, 3)).reshape(_BATCH, _SEQLEN, _EMB)


# speedup vs baseline: 2.3030x; 1.0013x over previous
import functools

import jax
import jax.numpy as jnp
from jax import lax
from jax.experimental import pallas as pl
from jax.experimental.pallas import tpu as pltpu
from jax.experimental.pallas import tpu_sc as plsc

_VOCAB = 100000
_EMB = 64
_BATCH = 1024
_SEQLEN = 200

_NC = 2
_NS = 16
_NW = _NC * _NS
_NBLK = _BATCH // 128
_SPW = _SEQLEN * _NBLK // _NW
_CH = 128


def _pos_plus_seg(seg_table):
    pos = jnp.arange(_SEQLEN, dtype=jnp.float32)[:, None]
    d = jnp.arange(_EMB, dtype=jnp.float32)
    d = 2.0 * d / _EMB
    p = pos / jnp.power(10000.0, d)
    p = p.at[:, ::2].set(jnp.sin(p[:, ::2]))
    p = p.at[:, 1::2].set(jnp.cos(p[:, 1::2]))
    seg_sel = (jnp.arange(_SEQLEN) >= _SEQLEN // 2 + 1)[:, None]
    seg = jnp.where(seg_sel, seg_table[1][None, :], seg_table[0][None, :])
    return p + seg


_GATHER_DNUMS = lax.GatherDimensionNumbers(
    offset_dims=(), collapsed_slice_dims=(0,), start_index_map=(0,))


def _bfly_perms():
    lane = lax.iota(jnp.int32, 16)
    return [(lane ^ k).reshape(16, 1) for k in (1, 2, 4, 8)]


def _sc_body(idx_hbm, table_hbm, cmb_hbm, g_hbm, b_hbm, out_hbm,
             idx_v, rows0, rows1, ob0, ob1, cmb_v, g_v, b_v,
             sg0, sg1, so0, so1):
    wid = lax.axis_index("s") * _NC + lax.axis_index("c")
    blk = lax.rem(wid, _NBLK)
    s0 = lax.div(wid, _NBLK) * _SPW
    pltpu.sync_copy(cmb_hbm.at[pl.ds(s0, _SPW)], cmb_v)
    pltpu.sync_copy(g_hbm, g_v)
    pltpu.sync_copy(b_hbm, b_v)
    pltpu.sync_copy(idx_hbm.at[pl.ds(s0, _SPW)], idx_v)
    rows = [rows0, rows1]
    outs = [ob0, ob1]
    sg = [sg0, sg1]
    so = [so0, so1]

    g_regs = [g_v[pl.ds(16 * j, 16)] for j in range(4)]
    b_regs = [b_v[pl.ds(16 * j, 16)] for j in range(4)]
    perms = _bfly_perms()
    lane = lax.iota(jnp.int32, 16)
    tmasks = [lax.bitwise_and(lane, t) == 0 for t in (1, 2, 4, 8)]

    def transpose16(regs):
        for ti, t in enumerate((1, 2, 4, 8)):
            m = tmasks[ti]
            new = []
            for k in range(16):
                other = lax.gather(
                    regs[k ^ t], perms[ti], dimension_numbers=_GATHER_DNUMS,
                    slice_sizes=(1,),
                    mode=lax.GatherScatterMode.PROMISE_IN_BOUNDS)
                if k & t == 0:
                    new.append(jnp.where(m, regs[k], other))
                else:
                    new.append(jnp.where(m, other, regs[k]))
            regs = new
        return regs

    def compute(c, rows_v, out_b):
        cmb_regs = [cmb_v[c, pl.ds(16 * j, 16)] for j in range(4)]

        def row_body(i, rcarry):
            acc_s = None
            acc_q = None
            for jj in range(4):
                regs = []
                for k in range(16):
                    r = i * 16 + k
                    regs.append(rows_v[r, pl.ds(16 * jj, 16)] + cmb_regs[jj])
                regs = transpose16(regs)
                for l in range(16):
                    t = regs[l]
                    if acc_s is None:
                        acc_s, acc_q = t, t * t
                    else:
                        acc_s = acc_s + t
                        acc_q = acc_q + t * t
                    out_b[pl.ds((16 * jj + l) * 128 + i * 16, 16)] = t
            mean = acc_s * (1.0 / 64.0)
            varv = acc_q * (1.0 / 64.0) - mean * mean + 1e-5
            iv = lax.bitcast_convert_type(varv, jnp.int32)
            y = lax.bitcast_convert_type(
                jnp.int32(0x5F375A86) - lax.shift_right_logical(iv, 1),
                jnp.float32)
            xh = varv * 0.5
            for _ in range(2):
                y = y * (1.5 - xh * y * y)
            off = mean * y
            for jj in range(4):
                for l in range(16):
                    j = 16 * jj + l
                    t = out_b[pl.ds(j * 128 + i * 16, 16)]
                    out_b[pl.ds(j * 128 + i * 16, 16)] = (
                        (t * y - off) * g_regs[jj][l] + b_regs[jj][l])
            return rcarry

        lax.fori_loop(0, _CH // 16, row_body, 0)

    def do_chunk(c, b, prefetch, outwait):
        pltpu.make_async_copy(
            table_hbm.at[idx_v.at[c, blk]], rows[b], sg[b]).wait()
        if prefetch is not None:
            pltpu.async_copy(
                table_hbm.at[idx_v.at[prefetch, blk]], rows[1 - b],
                sg[1 - b])
        off = ((s0 + c) * 8 * _NBLK + blk) * 1024
        if outwait:
            for tr in range(8):
                pltpu.make_async_copy(
                    outs[b].at[pl.ds(tr * 1024, 1024)],
                    out_hbm.at[pl.ds(off + tr * _NBLK * 1024, 1024)],
                    so[b]).wait()
        compute(c, rows[b], outs[b])
        for tr in range(8):
            pltpu.async_copy(
                outs[b].at[pl.ds(tr * 1024, 1024)],
                out_hbm.at[pl.ds(off + tr * _NBLK * 1024, 1024)], so[b])

    pltpu.async_copy(table_hbm.at[idx_v.at[0, blk]], rows[0], sg[0])
    do_chunk(0, 0, 1, False)
    do_chunk(1, 1, 2, False)

    def grp(g, carry):
        c0 = 2 * g + 2
        do_chunk(c0, 0, c0 + 1, True)
        do_chunk(c0 + 1, 1, c0 + 2, True)
        return carry

    lax.fori_loop(0, (_SPW - 4) // 2, grp, 0)
    do_chunk(_SPW - 2, 0, _SPW - 1, True)
    do_chunk(_SPW - 1, 1, None, True)
    for b, c in ((0, _SPW - 2), (1, _SPW - 1)):
        off = ((s0 + c) * 8 * _NBLK + blk) * 1024
        for tr in range(8):
            pltpu.make_async_copy(
                outs[b].at[pl.ds(tr * 1024, 1024)],
                out_hbm.at[pl.ds(off + tr * _NBLK * 1024, 1024)],
                so[b]).wait()


def kernel(seq, token_table, seg_table, gamma, beta):
    cmb = _pos_plus_seg(seg_table)
    idx3 = jnp.transpose(seq, (1, 0)).reshape(_SEQLEN, _NBLK, 128)
    run = functools.partial(
        pl.kernel,
        out_type=jax.ShapeDtypeStruct((_SEQLEN * 8 * _NBLK * 8 * 128,),
                                      jnp.float32),
        mesh=plsc.VectorSubcoreMesh(core_axis_name="c", subcore_axis_name="s"),
        scratch_types=[
            pltpu.VMEM((_SPW, _NBLK, 128), jnp.int32),
            pltpu.VMEM((_CH, _EMB), jnp.float32),
            pltpu.VMEM((_CH, _EMB), jnp.float32),
            pltpu.VMEM((8192,), jnp.float32),
            pltpu.VMEM((8192,), jnp.float32),
            pltpu.VMEM((_SPW, _EMB), jnp.float32),
            pltpu.VMEM((_EMB,), jnp.float32),
            pltpu.VMEM((_EMB,), jnp.float32),
            pltpu.SemaphoreType.DMA,
            pltpu.SemaphoreType.DMA,
            pltpu.SemaphoreType.DMA,
            pltpu.SemaphoreType.DMA,
        ],
        compiler_params=pltpu.CompilerParams(use_tc_tiling_on_sc=False),
    )(_sc_body)
    outf = run(idx3, token_table, cmb, gamma, beta)
    out5 = outf.reshape(_SEQLEN, 8, _NBLK, 8, 128)
    return out5.transpose((2, 4, 0, 1, 3)).reshape(_BATCH, _SEQLEN, _EMB)
